# Initial kernel scaffold; baseline (speedup 1.0000x reference)
import jax
import jax.numpy as jnp
from jax.experimental import pallas as pl


def kernel(z, action, edge_index, W1, b1, W2, b2):
    raise NotImplementedError("write your pallas kernel here")


# speedup vs baseline: 19.7879x; 19.7879x over previous
import functools

import jax
import jax.numpy as jnp
from jax import lax
from jax.experimental import pallas as pl
from jax.experimental.pallas import tpu as pltpu
from jax.experimental.pallas import tpu_sc as plsc

N_NODES = 100000
N_EDGES = 1600000
LANES = 16
NC = 2
NS = 16
K_CHUNK = 2000

ROWS_PER_TILE = ((N_NODES + NS - 1) // NS + 7) // 8 * 8
N_ALLOC = ROWS_PER_TILE * NS

_SC_PARAMS = pltpu.CompilerParams(use_tc_tiling_on_sc=False)
_SC_MESH = dict(core_axis_name="c", subcore_axis_name="s")


def _fill_1d(ref, size, value):
  vec = jnp.full((LANES,), value, dtype=ref.dtype)

  def body(i, _):
    ref[pl.ds(i * LANES, LANES)] = vec
    return 0

  lax.fori_loop(0, size // LANES, body, 0)


def _fill_2d(ref, rows, value):
  vec = jnp.full((LANES,), value, dtype=ref.dtype)

  def body(i, _):
    ref[i, :] = vec
    return 0

  lax.fori_loop(0, rows, body, 0)


_EDGES_PER_TILE_H = N_EDGES // (NC * NS)


def _hist_body(dst_hbm, out_hbm, acc_sh, ones_v, didx_v, zbuf_v):
  cid = lax.axis_index("c")
  sid = lax.axis_index("s")

  _fill_1d(ones_v, K_CHUNK, 1.0)
  _fill_1d(zbuf_v, ROWS_PER_TILE, 0.0)
  pltpu.sync_copy(zbuf_v, acc_sh.at[pl.ds(sid * ROWS_PER_TILE, ROWS_PER_TILE)])
  plsc.subcore_barrier()

  base = (cid * NS + sid) * _EDGES_PER_TILE_H

  def body(i, _):
    off = base + i * K_CHUNK
    pltpu.sync_copy(dst_hbm.at[pl.ds(off, K_CHUNK)], didx_v)
    pltpu.sync_copy(ones_v, acc_sh.at[didx_v], add=True)
    return 0

  lax.fori_loop(0, _EDGES_PER_TILE_H // K_CHUNK, body, 0)
  plsc.subcore_barrier()

  r0 = sid * ROWS_PER_TILE
  pltpu.sync_copy(acc_sh.at[pl.ds(r0, ROWS_PER_TILE)], zbuf_v)
  pltpu.sync_copy(zbuf_v, out_hbm.at[pl.ds(cid * N_ALLOC + r0, ROWS_PER_TILE)])


_hist_call = pl.kernel(
    _hist_body,
    out_type=jax.ShapeDtypeStruct((NC * N_ALLOC,), jnp.float32),
    mesh=plsc.VectorSubcoreMesh(**_SC_MESH),
    compiler_params=_SC_PARAMS,
    scratch_types=[
        pltpu.VMEM_SHARED((N_ALLOC,), jnp.float32),
        pltpu.VMEM((K_CHUNK,), jnp.float32),
        pltpu.VMEM((K_CHUNK,), jnp.int32),
        pltpu.VMEM((ROWS_PER_TILE,), jnp.float32),
    ],
)


_EDGES_PER_TILE_P = N_EDGES // NS
KP_CHUNK = 1000
_ZROWS = ROWS_PER_TILE // LANES
_OCHUNK = ROWS_PER_TILE // 8


def _prop_body(n_slabs, *refs):
  src_hbm, dst_hbm = refs[0], refs[1]
  ys = refs[2:2 + n_slabs]
  outs = refs[2 + n_slabs:2 + 2 * n_slabs]
  acc_sh, sidx_v, didx_v, rows_v, zbuf_v = refs[2 + 2 * n_slabs:]

  passes_per_core = n_slabs // NC
  cid = lax.axis_index("c")
  sid = lax.axis_index("s")

  _fill_2d(zbuf_v, _ZROWS, 0.0)
  base = sid * _EDGES_PER_TILE_P

  for c in range(NC):

    @pl.when(cid == c)
    def _():
      for pp in range(passes_per_core):
        p = c * passes_per_core + pp
        y_hbm = ys[p]
        out_hbm = outs[p]

        def zbody(j, _):
          pltpu.sync_copy(
              zbuf_v,
              acc_sh.at[pl.ds(sid * ROWS_PER_TILE + j * _ZROWS, _ZROWS)])
          return 0

        lax.fori_loop(0, LANES, zbody, 0)
        plsc.subcore_barrier()

        def body(i, _):
          off = base + i * KP_CHUNK
          pltpu.sync_copy(src_hbm.at[pl.ds(off, KP_CHUNK)], sidx_v)
          pltpu.sync_copy(dst_hbm.at[pl.ds(off, KP_CHUNK)], didx_v)
          pltpu.sync_copy(y_hbm.at[sidx_v], rows_v)
          pltpu.sync_copy(rows_v, acc_sh.at[didx_v], add=True)
          return 0

        lax.fori_loop(0, _EDGES_PER_TILE_P // KP_CHUNK, body, 0)
        plsc.subcore_barrier()

        r0 = sid * ROWS_PER_TILE

        def obody(j, _):
          rstart = r0 + j * _OCHUNK
          pltpu.sync_copy(acc_sh.at[pl.ds(rstart, _OCHUNK)],
                          rows_v.at[pl.ds(0, _OCHUNK)])
          pltpu.sync_copy(rows_v.at[pl.ds(0, _OCHUNK)],
                          out_hbm.at[pl.ds(rstart, _OCHUNK)])
          return 0

        lax.fori_loop(0, 8, obody, 0)


def _make_prop(n_slabs):
  return pl.kernel(
      functools.partial(_prop_body, n_slabs),
      out_type=[jax.ShapeDtypeStruct((N_ALLOC, LANES), jnp.float32)
                for _ in range(n_slabs)],
      mesh=plsc.VectorSubcoreMesh(**_SC_MESH),
      compiler_params=_SC_PARAMS,
      scratch_types=[
          pltpu.VMEM_SHARED((N_ALLOC, LANES), jnp.float32),
          pltpu.VMEM((KP_CHUNK,), jnp.int32),
          pltpu.VMEM((KP_CHUNK,), jnp.int32),
          pltpu.VMEM((KP_CHUNK, LANES), jnp.float32),
          pltpu.VMEM((_ZROWS, LANES), jnp.float32),
      ],
  )


_prop4 = _make_prop(4)
_prop2 = _make_prop(2)


_BN = 2000
_GRID = N_NODES // _BN


def _tc1_body(xp_ref, h0_ref, h1_ref, w_ref, y0_ref, y1_ref, y2_ref, y3_ref,
              dis_ref):
  deg = h0_ref[...] + h1_ref[...] + 1.0
  dis = lax.rsqrt(deg)
  xw = jnp.dot(xp_ref[...], w_ref[...],
               preferred_element_type=jnp.float32,
               precision=lax.Precision.HIGHEST)
  y = xw * dis
  y0_ref[...] = y[:, 0:16]
  y1_ref[...] = y[:, 16:32]
  y2_ref[...] = y[:, 32:48]
  y3_ref[...] = y[:, 48:64]
  dis_ref[...] = dis


def _tc1_call(xp, h0, h1, w1p):
  slab = pl.BlockSpec((_BN, LANES), lambda i: (i, 0))
  col = pl.BlockSpec((_BN, 1), lambda i: (i, 0))
  return pl.pallas_call(
      _tc1_body,
      grid=(_GRID,),
      in_specs=[
          pl.BlockSpec((_BN, 64), lambda i: (i, 0)),
          col, col,
          pl.BlockSpec((64, 64), lambda i: (0, 0)),
      ],
      out_specs=[slab, slab, slab, slab, col],
      out_shape=[jax.ShapeDtypeStruct((N_NODES, LANES), jnp.float32)] * 4
      + [jax.ShapeDtypeStruct((N_NODES, 1), jnp.float32)],
  )(xp, h0, h1, w1p)


def _tc2_body(a0, a1, a2, a3, y0, y1, y2, y3, dis_ref, b1_ref, w2_ref,
              o0_ref, o1_ref):
  acc = jnp.concatenate([a0[...], a1[...], a2[...], a3[...]], axis=1)
  y = jnp.concatenate([y0[...], y1[...], y2[...], y3[...]], axis=1)
  dis = dis_ref[...]
  h = jnp.maximum(dis * (acc + y) + b1_ref[...], 0.0)
  y2 = jnp.dot(h, w2_ref[...],
               preferred_element_type=jnp.float32,
               precision=lax.Precision.HIGHEST) * dis
  o0_ref[...] = y2[:, 0:16]
  o1_ref[...] = y2[:, 16:32]


def _tc2_call(accs, ys, dis, b1, w2):
  slab = pl.BlockSpec((_BN, LANES), lambda i: (i, 0))
  col = pl.BlockSpec((_BN, 1), lambda i: (i, 0))
  return pl.pallas_call(
      _tc2_body,
      grid=(_GRID,),
      in_specs=[slab] * 8 + [
          col,
          pl.BlockSpec((1, 64), lambda i: (0, 0)),
          pl.BlockSpec((64, 32), lambda i: (0, 0)),
      ],
      out_specs=[slab, slab],
      out_shape=[jax.ShapeDtypeStruct((N_NODES, LANES), jnp.float32)] * 2,
  )(*accs, *ys, dis, b1, w2)


def _tc3_body(a0, a1, y0, y1, dis_ref, b2_ref, out_ref):
  acc = jnp.concatenate([a0[...], a1[...]], axis=1)
  y = jnp.concatenate([y0[...], y1[...]], axis=1)
  out_ref[...] = dis_ref[...] * (acc + y) + b2_ref[...]


def _tc3_call(accs, ys, dis, b2):
  slab = pl.BlockSpec((_BN, LANES), lambda i: (i, 0))
  col = pl.BlockSpec((_BN, 1), lambda i: (i, 0))
  return pl.pallas_call(
      _tc3_body,
      grid=(_GRID,),
      in_specs=[slab] * 4 + [col, pl.BlockSpec((1, 32), lambda i: (0, 0))],
      out_specs=pl.BlockSpec((_BN, 32), lambda i: (i, 0)),
      out_shape=jax.ShapeDtypeStruct((N_NODES, 32), jnp.float32),
  )(*accs, *ys, dis, b2)


def kernel(z, action, edge_index, W1, b1, W2, b2):
  src = edge_index[0].astype(jnp.int32)
  dst = edge_index[1].astype(jnp.int32)

  xp = jnp.concatenate([z, action], axis=1)
  xp = jnp.pad(xp, ((0, 0), (0, 64 - xp.shape[1])))
  w1p = jnp.pad(W1, ((0, 64 - W1.shape[0]), (0, 0)))

  hist = _hist_call(dst)
  h0 = hist[:N_NODES].reshape(N_NODES, 1)
  h1 = hist[N_ALLOC:N_ALLOC + N_NODES].reshape(N_NODES, 1)

  *y1s, dis = _tc1_call(xp, h0, h1, w1p)
  acc1s = _prop4(src, dst, *y1s)
  y2s = _tc2_call(acc1s, y1s, dis, b1.reshape(1, -1), W2)
  acc2s = _prop2(src, dst, *y2s)
  return _tc3_call(acc2s, y2s, dis, b2.reshape(1, -1))


# speedup vs baseline: 24.1050x; 1.2182x over previous
import functools

import jax
import jax.numpy as jnp
from jax import lax
from jax.experimental import pallas as pl
from jax.experimental.pallas import tpu as pltpu
from jax.experimental.pallas import tpu_sc as plsc

N_NODES = 100000
N_EDGES = 1600000
LANES = 16
NC = 2
NS = 16
K_CHUNK = 2000

ROWS_PER_TILE = ((N_NODES + NS - 1) // NS + 7) // 8 * 8
N_ALLOC = ROWS_PER_TILE * NS

_SC_PARAMS = pltpu.CompilerParams(use_tc_tiling_on_sc=False)
_SC_MESH = dict(core_axis_name="c", subcore_axis_name="s")


def _fill_1d(ref, size, value):
  vec = jnp.full((LANES,), value, dtype=ref.dtype)

  def body(i, _):
    ref[pl.ds(i * LANES, LANES)] = vec
    return 0

  lax.fori_loop(0, size // LANES, body, 0)


def _fill_2d(ref, rows, value):
  vec = jnp.full((LANES,), value, dtype=ref.dtype)

  def body(i, _):
    ref[i, :] = vec
    return 0

  lax.fori_loop(0, rows, body, 0)


_EDGES_PER_TILE_H = N_EDGES // (NC * NS)


def _hist_body(dst_hbm, out_hbm, acc_sh, ones_v, didx_v, zbuf_v):
  cid = lax.axis_index("c")
  sid = lax.axis_index("s")

  _fill_1d(ones_v, K_CHUNK, 1.0)
  _fill_1d(zbuf_v, ROWS_PER_TILE, 0.0)
  pltpu.sync_copy(zbuf_v, acc_sh.at[pl.ds(sid * ROWS_PER_TILE, ROWS_PER_TILE)])
  plsc.subcore_barrier()

  base = (cid * NS + sid) * _EDGES_PER_TILE_H

  def body(i, _):
    off = base + i * K_CHUNK
    pltpu.sync_copy(dst_hbm.at[pl.ds(off, K_CHUNK)], didx_v)
    pltpu.sync_copy(ones_v, acc_sh.at[didx_v], add=True)
    return 0

  lax.fori_loop(0, _EDGES_PER_TILE_H // K_CHUNK, body, 0)
  plsc.subcore_barrier()

  r0 = sid * ROWS_PER_TILE
  pltpu.sync_copy(acc_sh.at[pl.ds(r0, ROWS_PER_TILE)], zbuf_v)
  pltpu.sync_copy(zbuf_v, out_hbm.at[pl.ds(cid * N_ALLOC + r0, ROWS_PER_TILE)])


_hist_call = pl.kernel(
    _hist_body,
    out_type=jax.ShapeDtypeStruct((NC * N_ALLOC,), jnp.float32),
    mesh=plsc.VectorSubcoreMesh(**_SC_MESH),
    compiler_params=_SC_PARAMS,
    scratch_types=[
        pltpu.VMEM_SHARED((N_ALLOC,), jnp.float32),
        pltpu.VMEM((K_CHUNK,), jnp.float32),
        pltpu.VMEM((K_CHUNK,), jnp.int32),
        pltpu.VMEM((ROWS_PER_TILE,), jnp.float32),
    ],
)


_EDGES_PER_TILE_P = N_EDGES // NS
KP_CHUNK = 1000
_ZROWS = ROWS_PER_TILE // LANES
_OCHUNK = ROWS_PER_TILE // 8


def _prop_body(n_slabs, *refs):
  srcs = refs[:n_slabs]
  dst_hbm = refs[n_slabs]
  y_hbm = refs[n_slabs + 1]
  out_hbm = refs[n_slabs + 2]
  acc_sh, sidx_v, didx_v, rows_v, zbuf_v = refs[n_slabs + 3:]

  passes_per_core = n_slabs // NC
  cid = lax.axis_index("c")
  sid = lax.axis_index("s")

  _fill_2d(zbuf_v, _ZROWS, 0.0)
  base = sid * _EDGES_PER_TILE_P

  for c in range(NC):

    @pl.when(cid == c)
    def _():
      for pp in range(passes_per_core):
        p = c * passes_per_core + pp
        src_hbm = srcs[p]

        def zbody(j, _):
          pltpu.sync_copy(
              zbuf_v,
              acc_sh.at[pl.ds(sid * ROWS_PER_TILE + j * _ZROWS, _ZROWS)])
          return 0

        lax.fori_loop(0, LANES, zbody, 0)
        plsc.subcore_barrier()

        def body(i, _):
          off = base + i * KP_CHUNK
          pltpu.sync_copy(src_hbm.at[pl.ds(off, KP_CHUNK)], sidx_v)
          pltpu.sync_copy(dst_hbm.at[pl.ds(off, KP_CHUNK)], didx_v)
          pltpu.sync_copy(y_hbm.at[sidx_v], rows_v)
          pltpu.sync_copy(rows_v, acc_sh.at[didx_v], add=True)
          return 0

        lax.fori_loop(0, _EDGES_PER_TILE_P // KP_CHUNK, body, 0)
        plsc.subcore_barrier()

        r0 = sid * ROWS_PER_TILE

        def obody(j, _):
          rstart = r0 + j * _OCHUNK
          pltpu.sync_copy(acc_sh.at[pl.ds(rstart, _OCHUNK)],
                          rows_v.at[pl.ds(0, _OCHUNK)])
          pltpu.sync_copy(rows_v.at[pl.ds(0, _OCHUNK)],
                          out_hbm.at[pl.ds(rstart, _OCHUNK),
                                     pl.ds(p * LANES, LANES)])
          return 0

        lax.fori_loop(0, 8, obody, 0)


def _make_prop(n_slabs):
  return pl.kernel(
      functools.partial(_prop_body, n_slabs),
      out_type=jax.ShapeDtypeStruct((N_ALLOC, n_slabs * LANES), jnp.float32),
      mesh=plsc.VectorSubcoreMesh(**_SC_MESH),
      compiler_params=_SC_PARAMS,
      scratch_types=[
          pltpu.VMEM_SHARED((N_ALLOC, LANES), jnp.float32),
          pltpu.VMEM((KP_CHUNK,), jnp.int32),
          pltpu.VMEM((KP_CHUNK,), jnp.int32),
          pltpu.VMEM((KP_CHUNK, LANES), jnp.float32),
          pltpu.VMEM((_ZROWS, LANES), jnp.float32),
      ],
  )


_prop4 = _make_prop(4)
_prop2 = _make_prop(2)


_GRID = 23
_BN = N_ALLOC // _GRID

_COL = pl.BlockSpec((_BN, 1), lambda i: (i, 0))


def _tc1_body(xp_ref, h0_ref, h1_ref, w_ref, y_ref, dis_ref):
  deg = h0_ref[...] + h1_ref[...] + 1.0
  dis = lax.rsqrt(deg)
  xw = jnp.dot(xp_ref[...], w_ref[...],
               preferred_element_type=jnp.float32,
               precision=lax.Precision.HIGHEST)
  y_ref[...] = xw * dis
  dis_ref[...] = dis


def _tc1_call(xp, h0, h1, w1p):
  return pl.pallas_call(
      _tc1_body,
      grid=(_GRID,),
      in_specs=[
          pl.BlockSpec((_BN, 64), lambda i: (i, 0)),
          _COL, _COL,
          pl.BlockSpec((64, 64), lambda i: (0, 0)),
      ],
      out_specs=[pl.BlockSpec((_BN, 64), lambda i: (i, 0)), _COL],
      out_shape=[jax.ShapeDtypeStruct((N_ALLOC, 64), jnp.float32),
                 jax.ShapeDtypeStruct((N_ALLOC, 1), jnp.float32)],
  )(xp, h0, h1, w1p)


def _tc2_body(acc_ref, y1_ref, dis_ref, b1_ref, w2_ref, y2_ref):
  dis = dis_ref[...]
  h = jnp.maximum(dis * (acc_ref[...] + y1_ref[...]) + b1_ref[...], 0.0)
  y2_ref[...] = jnp.dot(h, w2_ref[...],
                        preferred_element_type=jnp.float32,
                        precision=lax.Precision.HIGHEST) * dis


def _tc2_call(acc1, y1, dis, b1, w2):
  return pl.pallas_call(
      _tc2_body,
      grid=(_GRID,),
      in_specs=[
          pl.BlockSpec((_BN, 64), lambda i: (i, 0)),
          pl.BlockSpec((_BN, 64), lambda i: (i, 0)),
          _COL,
          pl.BlockSpec((1, 64), lambda i: (0, 0)),
          pl.BlockSpec((64, 32), lambda i: (0, 0)),
      ],
      out_specs=pl.BlockSpec((_BN, 32), lambda i: (i, 0)),
      out_shape=jax.ShapeDtypeStruct((N_ALLOC, 32), jnp.float32),
  )(acc1, y1, dis, b1, w2)


def _tc3_body(acc_ref, y2_ref, dis_ref, b2_ref, out_ref):
  out_ref[...] = dis_ref[...] * (acc_ref[...] + y2_ref[...]) + b2_ref[...]


def _tc3_call(acc2, y2, dis, b2):
  return pl.pallas_call(
      _tc3_body,
      grid=(_GRID,),
      in_specs=[
          pl.BlockSpec((_BN, 32), lambda i: (i, 0)),
          pl.BlockSpec((_BN, 32), lambda i: (i, 0)),
          _COL,
          pl.BlockSpec((1, 32), lambda i: (0, 0)),
      ],
      out_specs=pl.BlockSpec((_BN, 32), lambda i: (i, 0)),
      out_shape=jax.ShapeDtypeStruct((N_ALLOC, 32), jnp.float32),
  )(acc2, y2, dis, b2)


def kernel(z, action, edge_index, W1, b1, W2, b2):
  src = edge_index[0].astype(jnp.int32)
  dst = edge_index[1].astype(jnp.int32)
  src4 = [src * 4 + p for p in range(4)]
  src2 = [src * 2 + p for p in range(2)]

  xp = jnp.concatenate([z, action], axis=1)
  xp = jnp.pad(xp, ((0, N_ALLOC - N_NODES), (0, 64 - xp.shape[1])))
  w1p = jnp.pad(W1, ((0, 64 - W1.shape[0]), (0, 0)))

  hist = _hist_call(dst)
  h0 = hist[:N_ALLOC].reshape(N_ALLOC, 1)
  h1 = hist[N_ALLOC:].reshape(N_ALLOC, 1)

  y1, dis = _tc1_call(xp, h0, h1, w1p)
  acc1 = _prop4(*src4, dst, y1.reshape(4 * N_ALLOC, LANES))
  y2 = _tc2_call(acc1, y1, dis, b1.reshape(1, -1), W2)
  acc2 = _prop2(*src2, dst, y2.reshape(2 * N_ALLOC, LANES))
  out = _tc3_call(acc2, y2, dis, b2.reshape(1, -1))
  return out[:N_NODES]


# speedup vs baseline: 28.1867x; 1.1693x over previous
import functools

import jax
import jax.numpy as jnp
from jax import lax
from jax.experimental import pallas as pl
from jax.experimental.pallas import tpu as pltpu
from jax.experimental.pallas import tpu_sc as plsc

N_NODES = 100000
N_EDGES = 1600000
LANES = 16
NC = 2
NS = 16
K_CHUNK = 2000

ROWS_PER_TILE = ((N_NODES + NS - 1) // NS + 7) // 8 * 8
N_ALLOC = ROWS_PER_TILE * NS

_SC_PARAMS = pltpu.CompilerParams(use_tc_tiling_on_sc=False)
_SC_MESH = dict(core_axis_name="c", subcore_axis_name="s")


def _fill_1d(ref, size, value):
  vec = jnp.full((LANES,), value, dtype=ref.dtype)

  def body(i, _):
    ref[pl.ds(i * LANES, LANES)] = vec
    return 0

  lax.fori_loop(0, size // LANES, body, 0)


def _fill_2d(ref, rows, value):
  vec = jnp.full((LANES,), value, dtype=ref.dtype)

  def body(i, _):
    ref[i, :] = vec
    return 0

  lax.fori_loop(0, rows, body, 0)


_EDGES_PER_TILE_H = N_EDGES // (NC * NS)


def _hist_body(dst_hbm, out_hbm, acc_sh, ones_v, didx_v, zbuf_v):
  cid = lax.axis_index("c")
  sid = lax.axis_index("s")

  _fill_1d(ones_v, K_CHUNK, 1.0)
  _fill_1d(zbuf_v, ROWS_PER_TILE, 0.0)
  pltpu.sync_copy(zbuf_v, acc_sh.at[pl.ds(sid * ROWS_PER_TILE, ROWS_PER_TILE)])
  plsc.subcore_barrier()

  base = (cid * NS + sid) * _EDGES_PER_TILE_H

  def body(i, _):
    off = base + i * K_CHUNK
    pltpu.sync_copy(dst_hbm.at[pl.ds(off, K_CHUNK)], didx_v)
    pltpu.sync_copy(ones_v, acc_sh.at[didx_v], add=True)
    return 0

  lax.fori_loop(0, _EDGES_PER_TILE_H // K_CHUNK, body, 0)
  plsc.subcore_barrier()

  r0 = sid * ROWS_PER_TILE
  pltpu.sync_copy(acc_sh.at[pl.ds(r0, ROWS_PER_TILE)], zbuf_v)
  pltpu.sync_copy(zbuf_v, out_hbm.at[pl.ds(cid * N_ALLOC + r0, ROWS_PER_TILE)])


_hist_call = pl.kernel(
    _hist_body,
    out_type=jax.ShapeDtypeStruct((NC * N_ALLOC,), jnp.float32),
    mesh=plsc.VectorSubcoreMesh(**_SC_MESH),
    compiler_params=_SC_PARAMS,
    scratch_types=[
        pltpu.VMEM_SHARED((N_ALLOC,), jnp.float32),
        pltpu.VMEM((K_CHUNK,), jnp.float32),
        pltpu.VMEM((K_CHUNK,), jnp.int32),
        pltpu.VMEM((ROWS_PER_TILE,), jnp.float32),
    ],
)


KP_DATA = 500
KP_CHUNK = 504
_NCH = 200
_TILE_SPAN = _NCH * KP_CHUNK
_ZROWS = ROWS_PER_TILE // LANES
_OCHUNK = ROWS_PER_TILE // 8


def _prop_body(n_slabs, *refs):
  srcs = refs[:n_slabs]
  dst_hbm = refs[n_slabs]
  y_hbm = refs[n_slabs + 1]
  out_hbm = refs[n_slabs + 2]
  (acc_sh, si0, si1, si2, si3, di0, di1, di2, di3, rw0, rw1, zbuf_v,
   smi0, smi1, smi2, smi3, smg0, smg1, smsc0, smsc1) = refs[n_slabs + 3:]
  sidx = [si0, si1, si2, si3]
  didx = [di0, di1, di2, di3]
  rows = [rw0, rw1]
  semi = [smi0, smi1, smi2, smi3]
  semg = [smg0, smg1]
  semsc = [smsc0, smsc1]

  passes_per_core = n_slabs // NC
  cid = lax.axis_index("c")
  sid = lax.axis_index("s")

  _fill_2d(zbuf_v, _ZROWS, 0.0)
  base = sid * _TILE_SPAN

  for c in range(NC):

    @pl.when(cid == c)
    def _():
      for pp in range(passes_per_core):
        p = c * passes_per_core + pp
        src_hbm = srcs[p]

        def zbody(j, _):
          pltpu.sync_copy(
              zbuf_v,
              acc_sh.at[pl.ds(sid * ROWS_PER_TILE + j * _ZROWS, _ZROWS)])
          return 0

        lax.fori_loop(0, LANES, zbody, 0)
        plsc.subcore_barrier()

        def idx_start(g, t):
          off = base + g * KP_CHUNK
          pltpu.async_copy(src_hbm.at[pl.ds(off, KP_CHUNK)], sidx[t], semi[t])
          pltpu.async_copy(dst_hbm.at[pl.ds(off, KP_CHUNK)], didx[t], semi[t])

        def idx_wait(t):
          pltpu.make_async_copy(
              src_hbm.at[pl.ds(0, KP_CHUNK)], sidx[t], semi[t]).wait()
          pltpu.make_async_copy(
              dst_hbm.at[pl.ds(0, KP_CHUNK)], didx[t], semi[t]).wait()

        def gather_start(t, j):
          pltpu.async_copy(y_hbm.at[sidx[t]], rows[j], semg[j])

        def gather_wait(t, j):
          pltpu.make_async_copy(y_hbm.at[sidx[t]], rows[j], semg[j]).wait()

        def sc_start(t, j):
          pltpu.async_copy(rows[j], acc_sh.at[didx[t]], semsc[j], add=True)

        def sc_wait(t, j):
          pltpu.make_async_copy(rows[j], acc_sh.at[didx[t]], semsc[j]).wait()

        idx_start(0, 0)
        idx_wait(0); gather_start(0, 0); idx_start(1, 1)
        idx_wait(1); gather_start(1, 1); idx_start(2, 2)
        gather_wait(0, 0); sc_start(0, 0)
        sc_wait(0, 0); idx_wait(2); gather_start(2, 0); idx_start(3, 3)
        gather_wait(1, 1); sc_start(1, 1)
        sc_wait(1, 1); idx_wait(3); gather_start(3, 1); idx_start(4, 0)
        gather_wait(2, 0); sc_start(2, 0)

        def body(i, _):
          for jj in range(4):
            g = i * 4 + jj
            j = jj % 2
            t = jj
            tp = (jj - 1) % 4
            tn = (jj + 1) % 4
            sc_wait((jj + 2) % 4, j)
            idx_wait(t)
            gather_start(t, j)
            @pl.when(g + 1 < _NCH)
            def _():
              idx_start(g + 1, tn)
            gather_wait(tp, 1 - j)
            sc_start(tp, 1 - j)
          return 0

        lax.fori_loop(1, _NCH // 4, body, 0)

        gather_wait(3, 1); sc_start(3, 1)
        sc_wait(2, 0)
        sc_wait(3, 1)
        plsc.subcore_barrier()

        r0 = sid * ROWS_PER_TILE

        def obody(j, _):
          rstart = r0 + j * _OCHUNK
          pltpu.sync_copy(acc_sh.at[pl.ds(rstart, _OCHUNK)],
                          rw0.at[pl.ds(0, _OCHUNK)])
          pltpu.sync_copy(rw0.at[pl.ds(0, _OCHUNK)],
                          out_hbm.at[pl.ds(rstart, _OCHUNK),
                                     pl.ds(p * LANES, LANES)])
          return 0

        lax.fori_loop(0, 8, obody, 0)


def _make_prop(n_slabs):
  return pl.kernel(
      functools.partial(_prop_body, n_slabs),
      out_type=jax.ShapeDtypeStruct((N_ALLOC, n_slabs * LANES), jnp.float32),
      mesh=plsc.VectorSubcoreMesh(**_SC_MESH),
      compiler_params=_SC_PARAMS,
      scratch_types=[
          pltpu.VMEM_SHARED((N_ALLOC, LANES), jnp.float32),
      ] + [pltpu.VMEM((KP_CHUNK,), jnp.int32) for _ in range(8)] + [
          pltpu.VMEM((KP_CHUNK, LANES), jnp.float32),
          pltpu.VMEM((KP_CHUNK, LANES), jnp.float32),
          pltpu.VMEM((_ZROWS, LANES), jnp.float32),
      ] + [pltpu.SemaphoreType.DMA for _ in range(8)],
  )


_prop4 = _make_prop(4)
_prop2 = _make_prop(2)


_GRID = 23
_BN = N_ALLOC // _GRID

_COL = pl.BlockSpec((_BN, 1), lambda i: (i, 0))


def _tc1_body(xp_ref, h0_ref, h1_ref, w_ref, y_ref, dis_ref):
  deg = h0_ref[...] + h1_ref[...] + 1.0
  dis = lax.rsqrt(deg)
  xw = jnp.dot(xp_ref[...], w_ref[...],
               preferred_element_type=jnp.float32,
               precision=lax.Precision.HIGHEST)
  y_ref[...] = xw * dis
  dis_ref[...] = dis


def _tc1_call(xp, h0, h1, w1p):
  return pl.pallas_call(
      _tc1_body,
      grid=(_GRID,),
      in_specs=[
          pl.BlockSpec((_BN, 64), lambda i: (i, 0)),
          _COL, _COL,
          pl.BlockSpec((64, 64), lambda i: (0, 0)),
      ],
      out_specs=[pl.BlockSpec((_BN, 64), lambda i: (i, 0)), _COL],
      out_shape=[jax.ShapeDtypeStruct((N_ALLOC, 64), jnp.float32),
                 jax.ShapeDtypeStruct((N_ALLOC, 1), jnp.float32)],
  )(xp, h0, h1, w1p)


def _tc2_body(acc_ref, y1_ref, dis_ref, b1_ref, w2_ref, y2_ref):
  dis = dis_ref[...]
  h = jnp.maximum(dis * (acc_ref[...] + y1_ref[...]) + b1_ref[...], 0.0)
  y2_ref[...] = jnp.dot(h, w2_ref[...],
                        preferred_element_type=jnp.float32,
                        precision=lax.Precision.HIGHEST) * dis


def _tc2_call(acc1, y1, dis, b1, w2):
  return pl.pallas_call(
      _tc2_body,
      grid=(_GRID,),
      in_specs=[
          pl.BlockSpec((_BN, 64), lambda i: (i, 0)),
          pl.BlockSpec((_BN, 64), lambda i: (i, 0)),
          _COL,
          pl.BlockSpec((1, 64), lambda i: (0, 0)),
          pl.BlockSpec((64, 32), lambda i: (0, 0)),
      ],
      out_specs=pl.BlockSpec((_BN, 32), lambda i: (i, 0)),
      out_shape=jax.ShapeDtypeStruct((N_ALLOC, 32), jnp.float32),
  )(acc1, y1, dis, b1, w2)


def _tc3_body(acc_ref, y2_ref, dis_ref, b2_ref, out_ref):
  out_ref[...] = dis_ref[...] * (acc_ref[...] + y2_ref[...]) + b2_ref[...]


def _tc3_call(acc2, y2, dis, b2):
  return pl.pallas_call(
      _tc3_body,
      grid=(_GRID,),
      in_specs=[
          pl.BlockSpec((_BN, 32), lambda i: (i, 0)),
          pl.BlockSpec((_BN, 32), lambda i: (i, 0)),
          _COL,
          pl.BlockSpec((1, 32), lambda i: (0, 0)),
      ],
      out_specs=pl.BlockSpec((_BN, 32), lambda i: (i, 0)),
      out_shape=jax.ShapeDtypeStruct((N_ALLOC, 32), jnp.float32),
  )(acc2, y2, dis, b2)


def kernel(z, action, edge_index, W1, b1, W2, b2):
  src = edge_index[0].astype(jnp.int32)
  dst = edge_index[1].astype(jnp.int32)

  def chunkpad(a, fill):
    a2 = a.reshape(-1, KP_DATA)
    return jnp.pad(a2, ((0, 0), (0, KP_CHUNK - KP_DATA)),
                   constant_values=fill).reshape(-1)

  src4 = [chunkpad(src * 4 + p, 0) for p in range(4)]
  src2 = [chunkpad(src * 2 + p, 0) for p in range(2)]
  dstp = chunkpad(dst, N_ALLOC - 1)

  xp = jnp.concatenate([z, action], axis=1)
  xp = jnp.pad(xp, ((0, N_ALLOC - N_NODES), (0, 64 - xp.shape[1])))
  w1p = jnp.pad(W1, ((0, 64 - W1.shape[0]), (0, 0)))

  hist = _hist_call(dst)
  h0 = hist[:N_ALLOC].reshape(N_ALLOC, 1)
  h1 = hist[N_ALLOC:].reshape(N_ALLOC, 1)

  y1, dis = _tc1_call(xp, h0, h1, w1p)
  acc1 = _prop4(*src4, dstp, y1.reshape(4 * N_ALLOC, LANES))
  y2 = _tc2_call(acc1, y1, dis, b1.reshape(1, -1), W2)
  acc2 = _prop2(*src2, dstp, y2.reshape(2 * N_ALLOC, LANES))
  out = _tc3_call(acc2, y2, dis, b2.reshape(1, -1))
  return out[:N_NODES]


# speedup vs baseline: 30.2291x; 1.0725x over previous
import functools

import jax
import jax.numpy as jnp
from jax import lax
from jax.experimental import pallas as pl
from jax.experimental.pallas import tpu as pltpu
from jax.experimental.pallas import tpu_sc as plsc

N_NODES = 100000
N_EDGES = 1600000
LANES = 16
NC = 2
NS = 16
K_CHUNK = 2000

ROWS_PER_TILE = ((N_NODES + NS - 1) // NS + 7) // 8 * 8
N_ALLOC = ROWS_PER_TILE * NS

_SC_PARAMS = pltpu.CompilerParams(use_tc_tiling_on_sc=False)
_SC_MESH = dict(core_axis_name="c", subcore_axis_name="s")


def _fill_1d(ref, size, value):
  vec = jnp.full((LANES,), value, dtype=ref.dtype)

  def body(i, _):
    ref[pl.ds(i * LANES, LANES)] = vec
    return 0

  lax.fori_loop(0, size // LANES, body, 0)


def _fill_2d(ref, rows, value):
  vec = jnp.full((LANES,), value, dtype=ref.dtype)

  def body(i, _):
    ref[i, :] = vec
    return 0

  lax.fori_loop(0, rows, body, 0)


_EDGES_PER_TILE_H = N_EDGES // (NC * NS)


def _hist_body(dst_hbm, out_hbm, acc_sh, ones_v, didx_v, zbuf_v):
  cid = lax.axis_index("c")
  sid = lax.axis_index("s")

  _fill_1d(ones_v, K_CHUNK, 1.0)
  _fill_1d(zbuf_v, ROWS_PER_TILE, 0.0)
  pltpu.sync_copy(zbuf_v, acc_sh.at[pl.ds(sid * ROWS_PER_TILE, ROWS_PER_TILE)])
  plsc.subcore_barrier()

  base = (cid * NS + sid) * _EDGES_PER_TILE_H

  def body(i, _):
    off = base + i * K_CHUNK
    pltpu.sync_copy(dst_hbm.at[pl.ds(off, K_CHUNK)], didx_v)
    pltpu.sync_copy(ones_v, acc_sh.at[didx_v], add=True)
    return 0

  lax.fori_loop(0, _EDGES_PER_TILE_H // K_CHUNK, body, 0)
  plsc.subcore_barrier()

  r0 = sid * ROWS_PER_TILE
  pltpu.sync_copy(acc_sh.at[pl.ds(r0, ROWS_PER_TILE)], zbuf_v)
  pltpu.sync_copy(zbuf_v, out_hbm.at[pl.ds(cid * N_ALLOC + r0, ROWS_PER_TILE)])


_hist_call = pl.kernel(
    _hist_body,
    out_type=jax.ShapeDtypeStruct((NC * N_ALLOC,), jnp.float32),
    mesh=plsc.VectorSubcoreMesh(**_SC_MESH),
    compiler_params=_SC_PARAMS,
    scratch_types=[
        pltpu.VMEM_SHARED((N_ALLOC,), jnp.float32),
        pltpu.VMEM((K_CHUNK,), jnp.float32),
        pltpu.VMEM((K_CHUNK,), jnp.int32),
        pltpu.VMEM((ROWS_PER_TILE,), jnp.float32),
    ],
)


KP_DATA = 500
KP_CHUNK = 504
_NCH = 200
_TILE_SPAN = _NCH * KP_CHUNK
_ZROWS = ROWS_PER_TILE // LANES
_OCHUNK = ROWS_PER_TILE // 8


def _prop_body(n_slabs, *refs):
  src_hbm = refs[0]
  dst_hbm = refs[1]
  y_hbm = refs[2]
  out_hbm = refs[3]
  (acc_sh, si0, si1, si2, si3, di0, di1, di2, di3, rw0, rw1, zbuf_v,
   smi0, smi1, smi2, smi3, smg0, smg1, smsc0, smsc1) = refs[4:]
  sidx = [si0, si1, si2, si3]
  didx = [di0, di1, di2, di3]
  rows = [rw0, rw1]
  semi = [smi0, smi1, smi2, smi3]
  semg = [smg0, smg1]
  semsc = [smsc0, smsc1]

  passes_per_core = n_slabs // NC
  cid = lax.axis_index("c")
  sid = lax.axis_index("s")

  _fill_2d(zbuf_v, _ZROWS, 0.0)
  base = sid * _TILE_SPAN

  for c in range(NC):

    @pl.when(cid == c)
    def _():
      for pp in range(passes_per_core):
        p = c * passes_per_core + pp
        if p:
          y_ref = y_hbm.at[pl.ds(p, n_slabs * N_ALLOC - n_slabs)]
        else:
          y_ref = y_hbm

        def zbody(j, _):
          pltpu.sync_copy(
              zbuf_v,
              acc_sh.at[pl.ds(sid * ROWS_PER_TILE + j * _ZROWS, _ZROWS)])
          return 0

        lax.fori_loop(0, LANES, zbody, 0)
        plsc.subcore_barrier()

        def idx_start(g, t):
          off = base + g * KP_CHUNK
          pltpu.async_copy(src_hbm.at[pl.ds(off, KP_CHUNK)], sidx[t], semi[t])
          pltpu.async_copy(dst_hbm.at[pl.ds(off, KP_CHUNK)], didx[t], semi[t])

        def idx_wait(t):
          pltpu.make_async_copy(
              src_hbm.at[pl.ds(0, KP_CHUNK)], sidx[t], semi[t]).wait()
          pltpu.make_async_copy(
              dst_hbm.at[pl.ds(0, KP_CHUNK)], didx[t], semi[t]).wait()

        def gather_start(t, j):
          pltpu.async_copy(y_ref.at[sidx[t]], rows[j], semg[j])

        def gather_wait(t, j):
          pltpu.make_async_copy(y_ref.at[sidx[t]], rows[j], semg[j]).wait()

        def sc_start(t, j):
          pltpu.async_copy(rows[j], acc_sh.at[didx[t]], semsc[j], add=True)

        def sc_wait(t, j):
          pltpu.make_async_copy(rows[j], acc_sh.at[didx[t]], semsc[j]).wait()

        idx_start(0, 0)
        idx_wait(0); gather_start(0, 0); idx_start(1, 1)
        idx_wait(1); gather_start(1, 1); idx_start(2, 2)
        gather_wait(0, 0); sc_start(0, 0)
        sc_wait(0, 0); idx_wait(2); gather_start(2, 0); idx_start(3, 3)
        gather_wait(1, 1); sc_start(1, 1)
        sc_wait(1, 1); idx_wait(3); gather_start(3, 1); idx_start(4, 0)
        gather_wait(2, 0); sc_start(2, 0)

        def body(i, _):
          for jj in range(4):
            g = i * 4 + jj
            j = jj % 2
            t = jj
            tp = (jj - 1) % 4
            tn = (jj + 1) % 4
            sc_wait((jj + 2) % 4, j)
            idx_wait(t)
            gather_start(t, j)
            @pl.when(g + 1 < _NCH)
            def _():
              idx_start(g + 1, tn)
            gather_wait(tp, 1 - j)
            sc_start(tp, 1 - j)
          return 0

        lax.fori_loop(1, _NCH // 4, body, 0)

        gather_wait(3, 1); sc_start(3, 1)
        sc_wait(2, 0)
        sc_wait(3, 1)
        plsc.subcore_barrier()

        r0 = sid * ROWS_PER_TILE

        def obody(j, _):
          rstart = r0 + j * _OCHUNK
          pltpu.sync_copy(acc_sh.at[pl.ds(rstart, _OCHUNK)],
                          rw0.at[pl.ds(0, _OCHUNK)])
          pltpu.sync_copy(rw0.at[pl.ds(0, _OCHUNK)],
                          out_hbm.at[pl.ds(rstart, _OCHUNK),
                                     pl.ds(p * LANES, LANES)])
          return 0

        lax.fori_loop(0, 8, obody, 0)


def _make_prop(n_slabs):
  return pl.kernel(
      functools.partial(_prop_body, n_slabs),
      out_type=jax.ShapeDtypeStruct((N_ALLOC, n_slabs * LANES), jnp.float32),
      mesh=plsc.VectorSubcoreMesh(**_SC_MESH),
      compiler_params=_SC_PARAMS,
      scratch_types=[
          pltpu.VMEM_SHARED((N_ALLOC, LANES), jnp.float32),
      ] + [pltpu.VMEM((KP_CHUNK,), jnp.int32) for _ in range(8)] + [
          pltpu.VMEM((KP_CHUNK, LANES), jnp.float32),
          pltpu.VMEM((KP_CHUNK, LANES), jnp.float32),
          pltpu.VMEM((_ZROWS, LANES), jnp.float32),
      ] + [pltpu.SemaphoreType.DMA for _ in range(8)],
  )


_prop4 = _make_prop(4)
_prop2 = _make_prop(2)


_GRID = 23
_BN = N_ALLOC // _GRID

_COL = pl.BlockSpec((_BN, 1), lambda i: (i, 0))


def _tc1_body(xp_ref, h0_ref, h1_ref, w_ref, y_ref, dis_ref):
  deg = h0_ref[...] + h1_ref[...] + 1.0
  dis = lax.rsqrt(deg)
  xw = jnp.dot(xp_ref[...], w_ref[...],
               preferred_element_type=jnp.float32,
               precision=lax.Precision.HIGHEST)
  y_ref[...] = xw * dis
  dis_ref[...] = dis


def _tc1_call(xp, h0, h1, w1p):
  return pl.pallas_call(
      _tc1_body,
      grid=(_GRID,),
      in_specs=[
          pl.BlockSpec((_BN, 64), lambda i: (i, 0)),
          _COL, _COL,
          pl.BlockSpec((64, 64), lambda i: (0, 0)),
      ],
      out_specs=[pl.BlockSpec((_BN, 64), lambda i: (i, 0)), _COL],
      out_shape=[jax.ShapeDtypeStruct((N_ALLOC, 64), jnp.float32),
                 jax.ShapeDtypeStruct((N_ALLOC, 1), jnp.float32)],
  )(xp, h0, h1, w1p)


def _tc2_body(acc_ref, y1_ref, dis_ref, b1_ref, w2_ref, y2_ref):
  dis = dis_ref[...]
  h = jnp.maximum(dis * (acc_ref[...] + y1_ref[...]) + b1_ref[...], 0.0)
  y2_ref[...] = jnp.dot(h, w2_ref[...],
                        preferred_element_type=jnp.float32,
                        precision=lax.Precision.HIGHEST) * dis


def _tc2_call(acc1, y1, dis, b1, w2):
  return pl.pallas_call(
      _tc2_body,
      grid=(_GRID,),
      in_specs=[
          pl.BlockSpec((_BN, 64), lambda i: (i, 0)),
          pl.BlockSpec((_BN, 64), lambda i: (i, 0)),
          _COL,
          pl.BlockSpec((1, 64), lambda i: (0, 0)),
          pl.BlockSpec((64, 32), lambda i: (0, 0)),
      ],
      out_specs=pl.BlockSpec((_BN, 32), lambda i: (i, 0)),
      out_shape=jax.ShapeDtypeStruct((N_ALLOC, 32), jnp.float32),
  )(acc1, y1, dis, b1, w2)


def _tc3_body(acc_ref, y2_ref, dis_ref, b2_ref, out_ref):
  out_ref[...] = dis_ref[...] * (acc_ref[...] + y2_ref[...]) + b2_ref[...]


def _tc3_call(acc2, y2, dis, b2):
  return pl.pallas_call(
      _tc3_body,
      grid=(_GRID,),
      in_specs=[
          pl.BlockSpec((_BN, 32), lambda i: (i, 0)),
          pl.BlockSpec((_BN, 32), lambda i: (i, 0)),
          _COL,
          pl.BlockSpec((1, 32), lambda i: (0, 0)),
      ],
      out_specs=pl.BlockSpec((_BN, 32), lambda i: (i, 0)),
      out_shape=jax.ShapeDtypeStruct((N_NODES, 32), jnp.float32),
  )(acc2, y2, dis, b2)


def kernel(z, action, edge_index, W1, b1, W2, b2):
  src = edge_index[0].astype(jnp.int32)
  dst = edge_index[1].astype(jnp.int32)

  def chunkpad(a, fill):
    a2 = a.reshape(-1, KP_DATA)
    return jnp.pad(a2, ((0, 0), (0, KP_CHUNK - KP_DATA)),
                   constant_values=fill).reshape(-1)

  srcm4 = chunkpad(src * 4, 0)
  srcm2 = chunkpad(src * 2, 0)
  dstp = chunkpad(dst, N_ALLOC - 1)

  xp = jnp.concatenate([z, action], axis=1)
  xp = jnp.pad(xp, ((0, N_ALLOC - N_NODES), (0, 64 - xp.shape[1])))
  w1p = jnp.pad(W1, ((0, 64 - W1.shape[0]), (0, 0)))

  hist = _hist_call(dst)
  h0 = hist[:N_ALLOC].reshape(N_ALLOC, 1)
  h1 = hist[N_ALLOC:].reshape(N_ALLOC, 1)

  y1, dis = _tc1_call(xp, h0, h1, w1p)
  acc1 = _prop4(srcm4, dstp, y1.reshape(4 * N_ALLOC, LANES))
  y2 = _tc2_call(acc1, y1, dis, b1.reshape(1, -1), W2)
  acc2 = _prop2(srcm2, dstp, y2.reshape(2 * N_ALLOC, LANES))
  return _tc3_call(acc2, y2, dis, b2.reshape(1, -1))
